# baseline (speedup 1.0000x reference)
import functools

import jax
import jax.numpy as jnp
from jax import lax
from jax.experimental import pallas as pl
from jax.experimental.pallas import tpu as pltpu
from jax.experimental.pallas import tpu_sc as plsc

MAXLEN = 200
EMBED_DIM = 32
BATCH = 4096
VOCAB = 1000000

NC, NS = 2, 16
NW = NC * NS
TOTAL_ROWS = BATCH * MAXLEN
CHUNK = 128
UNITS = TOTAL_ROWS // CHUNK
UNITS_PER_W = UNITS // NW
BBLK = BATCH // CHUNK
NBUF = 4

TBLK_FULL = VOCAB // CHUNK
TAIL = VOCAB - TBLK_FULL * CHUNK
TBASE_EXTRA = TBLK_FULL - (TBLK_FULL // NW) * NW
TPW = TBLK_FULL // NW
SCR_ROWS = VOCAB * EMBED_DIM // 128

_MESH = plsc.VectorSubcoreMesh(core_axis_name="c", subcore_axis_name="s")


@functools.partial(
    pl.kernel,
    out_type=jax.ShapeDtypeStruct((SCR_ROWS, 128), jnp.float32),
    mesh=_MESH,
    compiler_params=pltpu.CompilerParams(use_tc_tiling_on_sc=True, needs_layout_passes=False),
    scratch_types=[
        pltpu.VMEM((NBUF, EMBED_DIM, CHUNK), jnp.float32),
        pltpu.VMEM((NBUF, 32, 128), jnp.float32),
        pltpu.VMEM((TAIL, 128), jnp.float32),
        pltpu.SemaphoreType.DMA((NBUF,)),
        pltpu.SemaphoreType.DMA((NBUF,)),
    ],
)
def _transpose_table(tt_hbm, tail_hbm, scr_hbm, tbuf, obuf, tailv, tsem, osem):
    wid = lax.axis_index("s") * NC + lax.axis_index("c")
    base = wid * TPW + lax.min(wid, TBASE_EXTRA)
    cnt = TPW + jnp.where(wid < TBASE_EXTRA, 1, 0)
    iota = lax.iota(jnp.int32, 16)

    def start_in(j, n):
        pltpu.make_async_copy(
            tt_hbm.at[:, pl.ds(j * CHUNK, CHUNK)], tbuf.at[n], tsem.at[n]
        ).start()

    for n in range(NBUF):
        @pl.when(n < cnt)
        def _():
            start_in(base + n, n)

    def block_body(t, _):
        j = base + t
        n = lax.rem(t, NBUF)
        pltpu.make_async_copy(
            tt_hbm.at[:, pl.ds(j * CHUNK, CHUNK)], tbuf.at[n], tsem.at[n]
        ).wait()

        @pl.when(t >= NBUF)
        def _():
            pltpu.make_async_copy(
                obuf.at[n], scr_hbm.at[pl.ds(0, 32)], osem.at[n]
            ).wait()

        def row_body(r, _):
            g0 = plsc.load_gather(tbuf.at[n], [iota, jnp.full((16,), r, jnp.int32)])
            g1 = plsc.load_gather(
                tbuf.at[n], [iota + 16, jnp.full((16,), r, jnp.int32)]
            )
            q, s = lax.div(r, 4), lax.rem(r, 4)
            obuf[n, q, pl.ds(s * 32, 16)] = g0
            obuf[n, q, pl.ds(s * 32 + 16, 16)] = g1
            return 0

        lax.fori_loop(0, CHUNK, row_body, 0, unroll=4)

        pltpu.make_async_copy(
            obuf.at[n], scr_hbm.at[pl.ds(j * 32, 32)], osem.at[n]
        ).start()

        @pl.when(t + NBUF < cnt)
        def _():
            start_in(j + NBUF, n)

        return 0

    lax.fori_loop(0, cnt, block_body, 0)
    for n in range(NBUF):
        @pl.when(n < cnt)
        def _():
            pltpu.make_async_copy(
                obuf.at[n], scr_hbm.at[pl.ds(0, 32)], osem.at[n]
            ).wait()

    @pl.when(wid == NW - 1)
    def _():
        pltpu.sync_copy(tail_hbm, tailv)

        def tail_q(q, _):
            for k in range(4):
                obuf[0, q, pl.ds(k * 32, 16)] = tailv[q * 4 + k, pl.ds(0, 16)]
                obuf[0, q, pl.ds(k * 32 + 16, 16)] = tailv[q * 4 + k, pl.ds(16, 16)]
            return 0

        lax.fori_loop(0, TAIL // 4, tail_q, 0)
        pltpu.sync_copy(
            obuf.at[0, pl.ds(0, TAIL // 4)],
            scr_hbm.at[pl.ds(TBLK_FULL * 32, TAIL // 4)],
        )


@functools.partial(
    pl.kernel,
    out_type=jax.ShapeDtypeStruct((UNITS * 4, 8, 128), jnp.float32),
    mesh=_MESH,
    compiler_params=pltpu.CompilerParams(use_tc_tiling_on_sc=False, needs_layout_passes=False),
    scratch_types=[
        pltpu.VMEM((UNITS_PER_W, CHUNK), jnp.int32),
        pltpu.VMEM((MAXLEN * EMBED_DIM,), jnp.float32),
        pltpu.VMEM((NBUF, CHUNK, EMBED_DIM), jnp.float32),
        pltpu.VMEM((NBUF, 4, 8, 128), jnp.float32),
        pltpu.SemaphoreType.DMA((NBUF,)),
        pltpu.SemaphoreType.DMA((NBUF,)),
    ],
)
def _gather_add(xt_hbm, scr_hbm, pos_hbm, out_hbm, idx_v, pos_v, gbuf, wbuf,
                gsem, wsem):
    wid = lax.axis_index("s") * NC + lax.axis_index("c")
    pltpu.sync_copy(xt_hbm.at[pl.ds(wid * UNITS_PER_W, UNITS_PER_W)], idx_v)
    pltpu.sync_copy(pos_hbm, pos_v)
    iota = lax.iota(jnp.int32, 16)
    ubase = wid * UNITS_PER_W

    def start_gather(t, n):
        pltpu.make_async_copy(
            scr_hbm.at[idx_v.at[t]], gbuf.at[n], gsem.at[n]
        ).start()

    for n in range(NBUF):
        start_gather(n, n)

    def unit_body(t, _):
        u = ubase + t
        n = lax.rem(t, NBUF)
        pltpu.make_async_copy(
            scr_hbm.at[idx_v.at[t]], gbuf.at[n], gsem.at[n]
        ).wait()

        @pl.when(t >= NBUF)
        def _():
            for i in range(4):
                pltpu.make_async_copy(
                    wbuf.at[n, i], out_hbm.at[0], wsem.at[n]
                ).wait()

        m = lax.div(u, BBLK)
        b = lax.rem(u, BBLK)
        m32 = m * EMBED_DIM

        def dim_body(d, _):
            posb = plsc.load_gather(pos_v, [jnp.full((16,), m32 + d, jnp.int32)])
            i, d8 = lax.div(d, 8), lax.rem(d, 8)
            for cv in range(8):
                g = plsc.load_gather(
                    gbuf.at[n], [iota + cv * 16, jnp.full((16,), d, jnp.int32)]
                )
                wbuf[n, i, d8, pl.ds(cv * 16, 16)] = g + posb
            return 0

        lax.fori_loop(0, EMBED_DIM, dim_body, 0, unroll=2)

        for i in range(4):
            pltpu.make_async_copy(
                wbuf.at[n, i], out_hbm.at[(m * 4 + i) * BBLK + b], wsem.at[n]
            ).start()

        @pl.when(t + NBUF < UNITS_PER_W)
        def _():
            start_gather(t + NBUF, n)

        return 0

    lax.fori_loop(0, UNITS_PER_W, unit_body, 0)
    for n in range(NBUF):
        for i in range(4):
            pltpu.make_async_copy(
                wbuf.at[n, i], out_hbm.at[0], wsem.at[n]
            ).wait()


def kernel(x, token_table, pos_table):
    tt = jnp.transpose(token_table)
    tail = jnp.pad(token_table[TBLK_FULL * CHUNK :], ((0, 0), (0, 128 - EMBED_DIM)))
    scr = _transpose_table(tt, tail)
    scr = scr.reshape(VOCAB, EMBED_DIM)
    xt = jnp.transpose(x).astype(jnp.int32).reshape(UNITS, CHUNK)
    pos_flat = pos_table.reshape(-1)
    out5 = _gather_add(xt, scr, pos_flat)
    out5 = out5.reshape(MAXLEN, 4, BBLK, 8, CHUNK)
    out = jnp.transpose(out5, (2, 4, 0, 1, 3)).reshape(BATCH, MAXLEN, EMBED_DIM)
    return out


# speedup vs baseline: 1.1548x; 1.1548x over previous
import functools

import jax
import jax.numpy as jnp
from jax import lax
from jax.experimental import pallas as pl
from jax.experimental.pallas import tpu as pltpu
from jax.experimental.pallas import tpu_sc as plsc

MAXLEN = 200
EMBED_DIM = 32
BATCH = 4096

NC, NS = 2, 16
NW = NC * NS
TOTAL_ROWS = BATCH * MAXLEN
ROWS_PER_W = TOTAL_ROWS // NW
CHUNK = 128
CHUNKS_PER_W = ROWS_PER_W // CHUNK
POS_PAD = MAXLEN + CHUNK
NBUF = 4


@functools.partial(
    pl.kernel,
    out_type=jax.ShapeDtypeStruct((TOTAL_ROWS, EMBED_DIM), jnp.float32),
    mesh=plsc.VectorSubcoreMesh(core_axis_name="c", subcore_axis_name="s"),
    compiler_params=pltpu.CompilerParams(use_tc_tiling_on_sc=False),
    scratch_types=[
        pltpu.VMEM((CHUNKS_PER_W, CHUNK), jnp.int32),
        pltpu.VMEM((POS_PAD, EMBED_DIM), jnp.float32),
        pltpu.VMEM((NBUF, CHUNK, EMBED_DIM), jnp.float32),
        pltpu.VMEM((NBUF, CHUNK, EMBED_DIM), jnp.float32),
        pltpu.SemaphoreType.DMA((NBUF,)),
        pltpu.SemaphoreType.DMA((NBUF,)),
    ],
)
def _emb(x_hbm, table_hbm, pos_hbm, out_hbm, idx_v, pos_v, gbuf, wbuf, gsem, wsem):
    wid = lax.axis_index("s") * NC + lax.axis_index("c")
    pltpu.sync_copy(x_hbm.at[pl.ds(wid * CHUNKS_PER_W, CHUNKS_PER_W)], idx_v)
    pltpu.sync_copy(pos_hbm, pos_v)
    out_base = wid * ROWS_PER_W

    def start_gather(c, b):
        pltpu.make_async_copy(
            table_hbm.at[idx_v.at[c]], gbuf.at[b], gsem.at[b]
        ).start()

    for b in range(NBUF):
        start_gather(b, b)

    def group(g, _):
        for b in range(NBUF):
            c = g * NBUF + b
            pltpu.make_async_copy(
                table_hbm.at[idx_v.at[c]], gbuf.at[b], gsem.at[b]
            ).wait()

            @pl.when(g > 0)
            def _():
                pltpu.make_async_copy(
                    wbuf.at[b], out_hbm.at[pl.ds(out_base, CHUNK)], wsem.at[b]
                ).wait()

            o = lax.rem(c * CHUNK, MAXLEN)

            def add_row(r, _):
                m = o + r
                wbuf[b, r, pl.ds(0, 16)] = (
                    gbuf[b, r, pl.ds(0, 16)] + pos_v[m, pl.ds(0, 16)]
                )
                wbuf[b, r, pl.ds(16, 16)] = (
                    gbuf[b, r, pl.ds(16, 16)] + pos_v[m, pl.ds(16, 16)]
                )
                return 0

            lax.fori_loop(0, CHUNK, add_row, 0, unroll=4)

            pltpu.make_async_copy(
                wbuf.at[b], out_hbm.at[pl.ds(out_base + c * CHUNK, CHUNK)], wsem.at[b]
            ).start()

            @pl.when(c + NBUF < CHUNKS_PER_W)
            def _():
                start_gather(c + NBUF, b)

        return 0

    lax.fori_loop(0, CHUNKS_PER_W // NBUF, group, 0)

    for b in range(NBUF):
        pltpu.make_async_copy(
            wbuf.at[b], out_hbm.at[pl.ds(out_base, CHUNK)], wsem.at[b]
        ).wait()


def kernel(x, token_table, pos_table):
    x_flat = x.reshape(-1).astype(jnp.int32)
    x2d = x_flat.reshape(NW * CHUNKS_PER_W, CHUNK)
    pos_ext = jnp.concatenate([pos_table, pos_table[: POS_PAD - MAXLEN]], axis=0)
    out = _emb(x2d, token_table, pos_ext)
    return out.reshape(BATCH, MAXLEN, EMBED_DIM)
